# baseline (speedup 1.0000x reference)
import functools

import jax
import jax.numpy as jnp
from jax import lax
from jax.experimental import pallas as pl
from jax.experimental.pallas import tpu as pltpu
from jax.experimental.pallas import tpu_sc as plsc

_N_TAGS = 64
_NUM_WORKERS = 32
_LANE = 128
_P = 2
_BW = 1024
_TILE = 8 * _LANE


@functools.lru_cache(maxsize=None)
def _make_gather(nb: int, nt: int):
    n_btiles = nb // _LANE
    n_gtiles = _N_TAGS // 8
    n_chunks = nt // _P
    assert n_btiles == _NUM_WORKERS and n_chunks % 2 == 0
    mesh = plsc.VectorSubcoreMesh(core_axis_name="c", subcore_axis_name="s")

    @functools.partial(
        pl.kernel,
        out_type=jax.ShapeDtypeStruct((nt, n_gtiles, n_btiles, _TILE),
                                      jnp.float32),
        mesh=mesh,
        scratch_types=[
            pltpu.VMEM((nt, _LANE), jnp.int32),
            pltpu.VMEM((2, _P, _LANE, _N_TAGS), jnp.float32),
            pltpu.VMEM((2, _P, _N_TAGS * _LANE), jnp.float32),
            pltpu.SemaphoreType.DMA,
            pltpu.SemaphoreType.DMA,
            pltpu.SemaphoreType.DMA,
            pltpu.SemaphoreType.DMA,
        ],
        compiler_params=pltpu.CompilerParams(use_tc_tiling_on_sc=False,
                                             needs_layout_passes=False),
    )
    def gather(table_hbm, idx_hbm, out_hbm, idx_v, rows_v, y_v, g0, g1, s0, s1):
        gsem = (g0, g1)
        ssem = (s0, s1)
        wid = lax.axis_index("s") * 2 + lax.axis_index("c")
        pltpu.sync_copy(
            idx_hbm.at[:, pl.ds(pl.multiple_of(wid * _LANE, _LANE), _LANE)],
            idx_v,
        )
        iota = lax.iota(jnp.int32, 16)
        rvecs = [iota + 16 * bb for bb in range(8)]

        def fire_gather(g, p):
            for tl in range(_P):
                pltpu.async_copy(
                    table_hbm.at[idx_v.at[g * _P + tl]],
                    rows_v.at[p, tl],
                    gsem[p],
                )

        def wait_gather(p):
            for tl in range(_P):
                pltpu.make_async_copy(
                    table_hbm.at[pl.ds(0, _LANE)], rows_v.at[p, tl], gsem[p]
                ).wait()

        def transpose_chunk(p):
            @plsc.parallel_loop(0, _N_TAGS, unroll=4)
            def body(ns):
                cvec = lax.broadcast(ns, (16,))
                for tl in range(_P):
                    for bb in range(8):
                        vals = plsc.load_gather(
                            rows_v.at[p, tl], [rvecs[bb], cvec]
                        )
                        y_v[p, tl, pl.ds(ns * _LANE + 16 * bb, 16)] = vals

        def fire_store(g, p):
            for tl in range(_P):
                for ng in range(n_gtiles):
                    pltpu.async_copy(
                        y_v.at[p, tl].at[pl.ds(ng * _TILE, _TILE)],
                        out_hbm.at[g * _P + tl, ng, wid],
                        ssem[p],
                    )

        def wait_store(p):
            for tl in range(_P):
                for ng in range(n_gtiles):
                    pltpu.make_async_copy(
                        y_v.at[p, tl].at[pl.ds(ng * _TILE, _TILE)],
                        out_hbm.at[0, 0, 0],
                        ssem[p],
                    ).wait()

        fire_gather(0, 0)

        def outer(i, carry):
            for b in range(2):
                g = i * 2 + b
                p = b
                q = 1 - b

                @pl.when(g + 1 < n_chunks)
                def _():
                    fire_gather(g + 1, q)

                wait_gather(p)

                @pl.when(g >= 2)
                def _():
                    wait_store(p)

                transpose_chunk(p)
                fire_store(g, p)
            return carry

        lax.fori_loop(0, n_chunks // 2, outer, 0)
        wait_store(0)
        wait_store(1)

    return gather


def _tpose_body(x_ref, o_ref):
    t = x_ref[...].T
    o_ref[...] = jnp.concatenate([t, t], axis=1)


def _pack_table(emits):
    n_tags, n_words = emits.shape
    grid = pl.cdiv(n_words, _BW)
    return pl.pallas_call(
        _tpose_body,
        grid=(grid,),
        in_specs=[pl.BlockSpec((n_tags, _BW), lambda i: (0, i))],
        out_specs=pl.BlockSpec((_BW, 2 * n_tags), lambda i: (i, 0)),
        out_shape=jax.ShapeDtypeStruct((n_words, 2 * n_tags), jnp.float32),
    )(emits)


def kernel(words, emits):
    nb, nt = words.shape
    n_tags = emits.shape[0]
    table = _pack_table(emits).reshape(-1, n_tags)
    idx_t = (words * 2).T
    out5 = _make_gather(nb, nt)(table, idx_t)
    out = (
        out5.reshape(nt, n_tags // 8, nb // _LANE, 8, _LANE)
        .transpose(2, 4, 0, 1, 3)
        .reshape(nb, nt, n_tags)
    )
    return out
